# Initial kernel scaffold; baseline (speedup 1.0000x reference)
import jax
import jax.numpy as jnp
from jax.experimental import pallas as pl


def kernel(discrete_x, continous_x, churn_date, edge_index, edge_attr, params):
    raise NotImplementedError("write your pallas kernel here")


# speedup vs baseline: 7.6714x; 7.6714x over previous
import jax
import jax.numpy as jnp
from jax import lax
from jax.experimental import pallas as pl
from jax.experimental.pallas import tpu as pltpu
from jax.experimental.pallas import tpu_sc as plsc

NN = 50000
EE = 800000
BT = 1024
NP = 49 * BT
NSC = 2
NTS = 16
RP = NP // NTS
BLK = 512
EP = 819200
EPT = EP // (NSC * NTS)
NBLK = EPT // BLK
NPAIR = NBLK // 2
CW = 16
DUMMY = NP - 1


def _lrelu(x):
    return jnp.where(x > 0, x, x * 0.01)


def _sc_mesh():
    return plsc.VectorSubcoreMesh(core_axis_name="c", subcore_axis_name="s")


def _sc_scratch():
    return [
        pltpu.VMEM((2, BLK), jnp.int32),
        pltpu.VMEM((2, BLK), jnp.int32),
        pltpu.VMEM((2, BLK), jnp.int32),
        pltpu.VMEM((2, BLK), jnp.int32),
        pltpu.VMEM((BLK, CW), jnp.float32),
        pltpu.VMEM((BLK, CW), jnp.float32),
        pltpu.VMEM_SHARED((NP, CW), jnp.float32),
        pltpu.SemaphoreType.DMA,
        pltpu.SemaphoreType.DMA,
        pltpu.SemaphoreType.DMA,
    ]


def _make_prop_body(C):
    def body(*refs):
        src2d, dst2d = refs[0], refs[1]
        tables = refs[2:2 + C]
        zeros_hbm = refs[2 + C]
        outs = refs[3 + C:3 + 2 * C]
        (sslab0, sslab1, dslab0, dslab1, rows0, rows1, acc,
         gsem0, gsem1, isem) = refs[3 + 2 * C:]
        sslab = (sslab0, sslab1)
        dslab = (dslab0, dslab1)
        rows = (rows0, rows1)
        gsem = (gsem0, gsem1)
        cid = lax.axis_index("c")
        sid = lax.axis_index("s")
        wid = cid * NTS + sid
        row0 = pl.multiple_of(sid * RP, 8)
        blk0 = wid * NBLK

        def load_slab(j, sp, sync):
            s = src2d.at[pl.ds(blk0 + 2 * j, 2)]
            d = dst2d.at[pl.ds(blk0 + 2 * j, 2)]
            if sync:
                pltpu.sync_copy(s, sslab[sp])
                pltpu.sync_copy(d, dslab[sp])
            else:
                pltpu.async_copy(s, sslab[sp], isem)
                pltpu.async_copy(d, dslab[sp], isem)

        def wait_slab(sp):
            pltpu.make_async_copy(src2d.at[pl.ds(blk0, 2)],
                                  sslab[sp], isem).wait()
            pltpu.make_async_copy(dst2d.at[pl.ds(blk0, 2)],
                                  dslab[sp], isem).wait()

        for ch in range(C):
            t = tables[ch]
            pltpu.sync_copy(zeros_hbm.at[pl.ds(row0, RP)],
                            acc.at[pl.ds(row0, RP)])
            plsc.subcore_barrier()
            load_slab(0, 0, True)
            pltpu.async_copy(t.at[sslab[0].at[0]], rows[0], gsem[0])

            def pair(j, sp, carry, _t=t, _last=False):
                sq = 1 - sp
                if not _last:
                    load_slab(j + 1, sq, False)
                pltpu.make_async_copy(_t.at[sslab[sp].at[0]],
                                      rows[0], gsem[0]).wait()
                pltpu.async_copy(_t.at[sslab[sp].at[1]], rows[1], gsem[1])
                pltpu.sync_copy(rows[0], acc.at[dslab[sp].at[0]], add=True)
                pltpu.make_async_copy(_t.at[sslab[sp].at[1]],
                                      rows[1], gsem[1]).wait()
                if not _last:
                    wait_slab(sq)
                    pltpu.async_copy(_t.at[sslab[sq].at[0]], rows[0], gsem[0])
                pltpu.sync_copy(rows[1], acc.at[dslab[sp].at[1]], add=True)
                return carry

            def pair2(j, carry, _t=t):
                pair(2 * j, 0, carry, _t)
                pair(2 * j + 1, 1, carry, _t)
                return carry

            lax.fori_loop(0, (NPAIR - 1) // 2, pair2, 0)
            pair(NPAIR - 1, (NPAIR - 1) % 2, 0, t, _last=True)
            plsc.subcore_barrier()
            pltpu.sync_copy(acc.at[pl.ds(row0, RP)],
                            outs[ch].at[cid].at[pl.ds(row0, RP)])
            plsc.subcore_barrier()
    return body


def _prop_call(src2d, dst2d, tables, zeros):
    C = len(tables)
    f = pl.kernel(
        _make_prop_body(C),
        out_type=[jax.ShapeDtypeStruct((NSC, NP, CW), jnp.float32)] * C,
        mesh=_sc_mesh(),
        scratch_types=_sc_scratch(),
        compiler_params=pltpu.CompilerParams(use_tc_tiling_on_sc=False),
    )
    out = f(src2d, dst2d, *tables, zeros)
    return out if isinstance(out, (list, tuple)) else [out]


def _dinv_of(deg2):
    deg = deg2[0, :, 0:1] + deg2[1, :, 0:1] + 1.0
    return lax.rsqrt(deg)


def _chunks(x):
    return [x[:, k * CW:(k + 1) * CW] for k in range(x.shape[1] // CW)]


def _pre_body(*refs):
    disc, churn, deg2, wg0, bg0, wf0, bf0, wn0, bn0 = refs[:9]
    outs = refs[9:]
    f32 = jnp.float32
    dinv = _dinv_of(deg2[...])
    d = disc[...]
    xg = _lrelu(jnp.dot(d, wg0[...], preferred_element_type=f32)
                + bg0[...]) * dinv
    xf = _lrelu(jnp.dot(d, wf0[...], preferred_element_type=f32)
                + bf0[...]) * dinv
    xn = _lrelu(jnp.dot(churn[...], wn0[...],
                        preferred_element_type=f32) + bn0[...]) * dinv
    for o, c in zip(outs, _chunks(xg) + _chunks(xf) + _chunks(xn)):
        o[...] = c


def _mid_body(*refs):
    aggs = refs[0:10]
    tabs = refs[10:20]
    deg2, wg1, bg1, wf1, bf1, wn1, bn1 = refs[20:27]
    outs = refs[27:]
    f32 = jnp.float32
    dinv = _dinv_of(deg2[...])
    s = [a[...][0] + a[...][1] + t[...] for a, t in zip(aggs, tabs)]
    tg = dinv * jnp.concatenate(s[0:4], axis=1)
    tf = dinv * jnp.concatenate(s[4:8], axis=1)
    tn = dinv * jnp.concatenate(s[8:10], axis=1)
    xg2 = _lrelu(jnp.dot(tg, wg1[...], preferred_element_type=f32)
                 + bg1[...]) * dinv
    xf2 = _lrelu(jnp.dot(tf, wf1[...], preferred_element_type=f32)
                 + bf1[...]) * dinv
    xn2 = _lrelu(jnp.dot(tn, wn1[...], preferred_element_type=f32)
                 + bn1[...])
    for o, c in zip(outs[:8], _chunks(xg2) + _chunks(xf2)):
        o[...] = c
    outs[8][...] = xn2


def _fin_body(*refs):
    disc, c1, c2 = refs[0:3]
    aggs = refs[3:11]
    tabs = refs[11:19]
    (deg2, xns2, wd, bd, wc1, bc1, wc2, bc2, wg2, bg2, wf2, bf2,
     fu0, fu1, fu2, fu3, bfu, wl1, bl1, wl2, bl2,
     wl3, bl3, wl4, bl4, rp) = refs[19:45]
    obig, osmall = refs[45:]
    f32 = jnp.float32
    dinv = _dinv_of(deg2[...])
    s = [a[...][0] + a[...][1] + t[...] for a, t in zip(aggs, tabs)]
    tg = dinv * jnp.concatenate(s[0:4], axis=1)
    tf = dinv * jnp.concatenate(s[4:8], axis=1)
    xg3 = _lrelu(jnp.dot(tg, wg2[...], preferred_element_type=f32) + bg2[...])
    xf3 = _lrelu(jnp.dot(tf, wf2[...], preferred_element_type=f32) + bf2[...])
    d = disc[...]
    xd = _lrelu(jnp.dot(d, wd[...], preferred_element_type=f32) + bd[...])
    xc1 = _lrelu(jnp.dot(c1[...], wc1[...], preferred_element_type=f32)
                 + bc1[...])
    xc2 = _lrelu(jnp.dot(c2[...], wc2[...], preferred_element_type=f32)
                 + bc2[...])
    h_ci = _lrelu(jnp.dot(xd, fu0[...], preferred_element_type=f32)
                  + jnp.dot(xc1, fu1[...], preferred_element_type=f32)
                  + jnp.dot(xc2, fu2[...], preferred_element_type=f32)
                  + jnp.dot(xg3, fu3[...], preferred_element_type=f32)
                  + bfu[...])
    h_si = xf3 * xns2[...]
    s_ci = jax.nn.sigmoid(
        jnp.dot(_lrelu(jnp.dot(h_ci, wl1[...], preferred_element_type=f32)
                       + bl1[...]), wl2[...], preferred_element_type=f32)
        + bl2[...])
    s_si = jax.nn.sigmoid(
        jnp.dot(_lrelu(jnp.dot(h_si, wl3[...], preferred_element_type=f32)
                       + bl3[...]), wl4[...], preferred_element_type=f32)
        + bl4[...])
    r = rp[...]
    r1a = _lrelu(s_ci * r[0:1, 0:1] + s_si * r[0:1, 1:2] + r[0:1, 2:3])
    r1b = _lrelu(s_ci * r[0:1, 3:4] + s_si * r[0:1, 4:5] + r[0:1, 5:6])
    y = jax.nn.sigmoid(r1a * r[0:1, 6:7] + r1b * r[0:1, 7:8] + r[0:1, 8:9])
    obig[...] = jnp.concatenate([h_ci, h_si], axis=1)
    z = jnp.zeros_like(y)
    osmall[...] = jnp.concatenate([y, s_ci, s_si, z, z, z, z, z], axis=1)


def _row_spec(cols):
    return pl.BlockSpec((BT, cols), lambda i: (i, 0))


def _full_spec(shape):
    nd = len(shape)
    return pl.BlockSpec(shape, lambda i: (0,) * nd)


def _agg_spec():
    return pl.BlockSpec((NSC, BT, CW), lambda i: (0, i, 0))


def _pad_rows(x):
    return jnp.pad(x, ((0, NP - x.shape[0]), (0, 0)))


def kernel(discrete_x, continous_x, churn_date, edge_index, edge_attr, params):
    p = params
    f32 = jnp.float32
    epad = jnp.full((EP - EE,), DUMMY, dtype=edge_index.dtype)
    src2d = jnp.concatenate([edge_index[0], epad]).reshape(EP // BLK, BLK)
    dst2d = jnp.concatenate([edge_index[1], epad]).reshape(EP // BLK, BLK)
    disc = _pad_rows(discrete_x)
    churn = _pad_rows(churn_date)
    c1 = _pad_rows(jnp.pad(continous_x[:, :13], ((0, 0), (0, 3))))
    c2 = _pad_rows(jnp.pad(continous_x[:, 13:26], ((0, 0), (0, 3))))
    zeros = jnp.zeros((NP, CW), f32)
    ones = jnp.ones((NP, CW), f32)

    def b2(b):
        return b.reshape(1, -1)

    deg2 = _prop_call(src2d, dst2d, [ones], zeros)[0]

    grid = (NP // BT,)
    table1 = pl.pallas_call(
        _pre_body,
        grid=grid,
        in_specs=[_row_spec(128), _row_spec(8), _agg_spec(),
                  _full_spec((128, 64)), _full_spec((1, 64)),
                  _full_spec((128, 64)), _full_spec((1, 64)),
                  _full_spec((8, 32)), _full_spec((1, 32))],
        out_specs=[_row_spec(CW)] * 10,
        out_shape=[jax.ShapeDtypeStruct((NP, CW), f32)] * 10,
    )(disc, churn, deg2, p["W_g0"], b2(p["b_g0"]), p["W_nf0"], b2(p["b_nf0"]),
      p["W_ns0"], b2(p["b_ns0"]))

    agg1 = _prop_call(src2d, dst2d, table1, zeros)

    table2_xn = pl.pallas_call(
        _mid_body,
        grid=grid,
        in_specs=[_agg_spec()] * 10 + [_row_spec(CW)] * 10 + [_agg_spec()] +
                 [_full_spec((64, 64)), _full_spec((1, 64)),
                  _full_spec((64, 64)), _full_spec((1, 64)),
                  _full_spec((32, 64)), _full_spec((1, 64))],
        out_specs=[_row_spec(CW)] * 8 + [_row_spec(64)],
        out_shape=[jax.ShapeDtypeStruct((NP, CW), f32)] * 8 +
                  [jax.ShapeDtypeStruct((NP, 64), f32)],
    )(*agg1, *table1, deg2, p["W_g1"], b2(p["b_g1"]),
      p["W_nf1"], b2(p["b_nf1"]), p["W_ns1"], b2(p["b_ns1"]))
    table2 = list(table2_xn[:8])
    xns2 = table2_xn[8]

    agg2 = _prop_call(src2d, dst2d, table2, zeros)

    rp = jnp.stack([
        p["W_r1"][0, 0], p["W_r1"][1, 0], p["b_r1"][0],
        p["W_r1"][0, 1], p["W_r1"][1, 1], p["b_r1"][1],
        p["W_r2"][0, 0], p["W_r2"][1, 0], p["b_r2"][0],
        jnp.float32(0), jnp.float32(0), jnp.float32(0), jnp.float32(0),
        jnp.float32(0), jnp.float32(0), jnp.float32(0)]).reshape(1, 16)

    wfu = p["W_fus"]
    big, small = pl.pallas_call(
        _fin_body,
        grid=grid,
        in_specs=[_row_spec(128), _row_spec(16), _row_spec(16)] +
                 [_agg_spec()] * 8 + [_row_spec(CW)] * 8 +
                 [_agg_spec(), _row_spec(64)] +
                 [_full_spec((128, 64)), _full_spec((1, 64)),
                  _full_spec((16, 64)), _full_spec((1, 64)),
                  _full_spec((16, 64)), _full_spec((1, 64)),
                  _full_spec((64, 64)), _full_spec((1, 64)),
                  _full_spec((64, 64)), _full_spec((1, 64)),
                  _full_spec((64, 64)), _full_spec((64, 64)),
                  _full_spec((64, 64)), _full_spec((64, 64)),
                  _full_spec((1, 64)),
                  _full_spec((64, 32)), _full_spec((1, 32)),
                  _full_spec((32, 1)), _full_spec((1, 1)),
                  _full_spec((64, 32)), _full_spec((1, 32)),
                  _full_spec((32, 1)), _full_spec((1, 1)),
                  _full_spec((1, 16))],
        out_specs=[_row_spec(128), _row_spec(8)],
        out_shape=[jax.ShapeDtypeStruct((NP, 128), f32),
                   jax.ShapeDtypeStruct((NP, 8), f32)],
    )(disc, c1, c2, *agg2, *table2, deg2, xns2,
      p["W_d"], b2(p["b_d"]),
      jnp.pad(p["W_c1"], ((0, 3), (0, 0))), b2(p["b_c1"]),
      jnp.pad(p["W_c2"], ((0, 3), (0, 0))), b2(p["b_c2"]),
      p["W_g2"], b2(p["b_g2"]), p["W_nf2"], b2(p["b_nf2"]),
      wfu[0:64], wfu[64:128], wfu[128:192], wfu[192:256], b2(p["b_fus"]),
      p["W_l1"], b2(p["b_l1"]), p["W_l2"], b2(p["b_l2"]),
      p["W_l3"], b2(p["b_l3"]), p["W_l4"], b2(p["b_l4"]), rp)

    y = small[:NN, 0]
    s_ci = small[:NN, 1:2]
    s_si = small[:NN, 2:3]
    h_ci = big[:NN, :64]
    h_si = big[:NN, 64:]
    return (y, s_ci, s_si, h_ci, h_si)


# speedup vs baseline: 8.8046x; 1.1477x over previous
import jax
import jax.numpy as jnp
from jax import lax
from jax.experimental import pallas as pl
from jax.experimental.pallas import tpu as pltpu
from jax.experimental.pallas import tpu_sc as plsc

NN = 50000
EE = 800000
BT = 1024
NP = 49 * BT
NSC = 2
NTS = 16
RP = NP // NTS
BLK = 1000
NBLK_TOT = EE // BLK
NB0 = 35
NB1 = 15
CW = 16


def _lrelu(x):
    return jnp.where(x > 0, x, x * 0.01)


def _sc_mesh():
    return plsc.VectorSubcoreMesh(core_axis_name="c", subcore_axis_name="s")


def _sc_scratch():
    return [
        pltpu.VMEM((BLK,), jnp.int32),
        pltpu.VMEM((BLK,), jnp.int32),
        pltpu.VMEM((BLK, CW), jnp.float32),
        pltpu.VMEM_SHARED((NP, CW), jnp.float32),
        pltpu.SemaphoreType.DMA,
    ]


def _make_prop_body(C):
    def body(*refs):
        src2d, dst2d = refs[0], refs[1]
        tables = refs[2:2 + C]
        zeros_hbm = refs[2 + C]
        outs = refs[3 + C:3 + 2 * C]
        sidx, didx, rows, acc, gsem = refs[3 + 2 * C:]
        cid = lax.axis_index("c")
        sid = lax.axis_index("s")
        row0 = pl.multiple_of(sid * RP, 8)
        blk0 = jnp.where(cid == 0, sid * NB0, NTS * NB0 + sid * NB1)
        nblk = jnp.where(cid == 0, NB0, NB1)
        for ch in range(C):
            t = tables[ch]
            pltpu.sync_copy(zeros_hbm.at[pl.ds(row0, RP)],
                            acc.at[pl.ds(row0, RP)])
            plsc.subcore_barrier()

            def blk(i, carry, _t=t):
                pltpu.sync_copy(src2d.at[blk0 + i], sidx)
                pltpu.sync_copy(dst2d.at[blk0 + i], didx)
                pltpu.async_copy(_t.at[sidx], rows, gsem).wait()
                pltpu.sync_copy(rows, acc.at[didx], add=True)
                return carry

            lax.fori_loop(0, nblk, blk, 0)
            plsc.subcore_barrier()
            pltpu.sync_copy(acc.at[pl.ds(row0, RP)],
                            outs[ch].at[cid].at[pl.ds(row0, RP)])
            plsc.subcore_barrier()
    return body


def _prop_call(src2d, dst2d, tables, zeros):
    C = len(tables)
    f = pl.kernel(
        _make_prop_body(C),
        out_type=[jax.ShapeDtypeStruct((NSC, NP, CW), jnp.float32)] * C,
        mesh=_sc_mesh(),
        scratch_types=_sc_scratch(),
        compiler_params=pltpu.CompilerParams(use_tc_tiling_on_sc=False),
    )
    out = f(src2d, dst2d, *tables, zeros)
    return out if isinstance(out, (list, tuple)) else [out]


def _dinv_of(deg2):
    deg = deg2[0, :, 0:1] + deg2[1, :, 0:1] + 1.0
    return lax.rsqrt(deg)


def _chunks(x):
    return [x[:, k * CW:(k + 1) * CW] for k in range(x.shape[1] // CW)]


def _pre_body(*refs):
    disc, churn, wg0, bg0, wf0, bf0, wn0, bn0 = refs[:8]
    outs = refs[8:]
    f32 = jnp.float32
    d = disc[...]
    xg = _lrelu(jnp.dot(d, wg0[...], preferred_element_type=f32) + bg0[...])
    xf = _lrelu(jnp.dot(d, wf0[...], preferred_element_type=f32) + bf0[...])
    xn = _lrelu(jnp.dot(churn[...], wn0[...],
                        preferred_element_type=f32) + bn0[...])
    for o, c in zip(outs, _chunks(xg) + _chunks(xf) + _chunks(xn)):
        o[...] = c


def _scale_body(*refs):
    tabs = refs[0:10]
    deg2 = refs[10]
    outs = refs[11:]
    dinv = _dinv_of(deg2[...])
    for o, t in zip(outs, tabs):
        o[...] = t[...] * dinv


def _mid_body(*refs):
    aggs = refs[0:10]
    tabs = refs[10:20]
    deg2, wg1, bg1, wf1, bf1, wn1, bn1 = refs[20:27]
    outs = refs[27:]
    f32 = jnp.float32
    dinv = _dinv_of(deg2[...])
    s = [a[...][0] + a[...][1] + t[...] for a, t in zip(aggs, tabs)]
    tg = dinv * jnp.concatenate(s[0:4], axis=1)
    tf = dinv * jnp.concatenate(s[4:8], axis=1)
    tn = dinv * jnp.concatenate(s[8:10], axis=1)
    xg2 = _lrelu(jnp.dot(tg, wg1[...], preferred_element_type=f32)
                 + bg1[...]) * dinv
    xf2 = _lrelu(jnp.dot(tf, wf1[...], preferred_element_type=f32)
                 + bf1[...]) * dinv
    xn2 = _lrelu(jnp.dot(tn, wn1[...], preferred_element_type=f32)
                 + bn1[...])
    for o, c in zip(outs[:8], _chunks(xg2) + _chunks(xf2)):
        o[...] = c
    outs[8][...] = xn2


def _fin_body(*refs):
    disc, c1, c2 = refs[0:3]
    aggs = refs[3:11]
    tabs = refs[11:19]
    (deg2, xns2, wd, bd, wc1, bc1, wc2, bc2, wg2, bg2, wf2, bf2,
     fu0, fu1, fu2, fu3, bfu, wl1, bl1, wl2, bl2,
     wl3, bl3, wl4, bl4, rp) = refs[19:45]
    obig, osmall = refs[45:]
    f32 = jnp.float32
    dinv = _dinv_of(deg2[...])
    s = [a[...][0] + a[...][1] + t[...] for a, t in zip(aggs, tabs)]
    tg = dinv * jnp.concatenate(s[0:4], axis=1)
    tf = dinv * jnp.concatenate(s[4:8], axis=1)
    xg3 = _lrelu(jnp.dot(tg, wg2[...], preferred_element_type=f32) + bg2[...])
    xf3 = _lrelu(jnp.dot(tf, wf2[...], preferred_element_type=f32) + bf2[...])
    d = disc[...]
    xd = _lrelu(jnp.dot(d, wd[...], preferred_element_type=f32) + bd[...])
    xc1 = _lrelu(jnp.dot(c1[...], wc1[...], preferred_element_type=f32)
                 + bc1[...])
    xc2 = _lrelu(jnp.dot(c2[...], wc2[...], preferred_element_type=f32)
                 + bc2[...])
    h_ci = _lrelu(jnp.dot(xd, fu0[...], preferred_element_type=f32)
                  + jnp.dot(xc1, fu1[...], preferred_element_type=f32)
                  + jnp.dot(xc2, fu2[...], preferred_element_type=f32)
                  + jnp.dot(xg3, fu3[...], preferred_element_type=f32)
                  + bfu[...])
    h_si = xf3 * xns2[...]
    s_ci = jax.nn.sigmoid(
        jnp.dot(_lrelu(jnp.dot(h_ci, wl1[...], preferred_element_type=f32)
                       + bl1[...]), wl2[...], preferred_element_type=f32)
        + bl2[...])
    s_si = jax.nn.sigmoid(
        jnp.dot(_lrelu(jnp.dot(h_si, wl3[...], preferred_element_type=f32)
                       + bl3[...]), wl4[...], preferred_element_type=f32)
        + bl4[...])
    r = rp[...]
    r1a = _lrelu(s_ci * r[0:1, 0:1] + s_si * r[0:1, 1:2] + r[0:1, 2:3])
    r1b = _lrelu(s_ci * r[0:1, 3:4] + s_si * r[0:1, 4:5] + r[0:1, 5:6])
    y = jax.nn.sigmoid(r1a * r[0:1, 6:7] + r1b * r[0:1, 7:8] + r[0:1, 8:9])
    obig[...] = jnp.concatenate([h_ci, h_si], axis=1)
    z = jnp.zeros_like(y)
    osmall[...] = jnp.concatenate([y, s_ci, s_si, z, z, z, z, z], axis=1)


def _row_spec(cols):
    return pl.BlockSpec((BT, cols), lambda i: (i, 0))


def _full_spec(shape):
    nd = len(shape)
    return pl.BlockSpec(shape, lambda i: (0,) * nd)


def _agg_spec():
    return pl.BlockSpec((NSC, BT, CW), lambda i: (0, i, 0))


def _pad_rows(x):
    return jnp.pad(x, ((0, NP - x.shape[0]), (0, 0)))


def kernel(discrete_x, continous_x, churn_date, edge_index, edge_attr, params):
    p = params
    f32 = jnp.float32
    src2d = edge_index[0].reshape(NBLK_TOT, BLK)
    dst2d = edge_index[1].reshape(NBLK_TOT, BLK)
    disc = _pad_rows(discrete_x)
    churn = _pad_rows(churn_date)
    c1 = _pad_rows(jnp.pad(continous_x[:, :13], ((0, 0), (0, 3))))
    c2 = _pad_rows(jnp.pad(continous_x[:, 13:26], ((0, 0), (0, 3))))
    zeros = jnp.zeros((NP, CW), f32)
    ones = jnp.ones((NP, CW), f32)

    def b2(b):
        return b.reshape(1, -1)

    deg2 = _prop_call(src2d, dst2d, [ones], zeros)[0]

    grid = (NP // BT,)
    table_u = pl.pallas_call(
        _pre_body,
        grid=grid,
        in_specs=[_row_spec(128), _row_spec(8),
                  _full_spec((128, 64)), _full_spec((1, 64)),
                  _full_spec((128, 64)), _full_spec((1, 64)),
                  _full_spec((8, 32)), _full_spec((1, 32))],
        out_specs=[_row_spec(CW)] * 10,
        out_shape=[jax.ShapeDtypeStruct((NP, CW), f32)] * 10,
    )(disc, churn, p["W_g0"], b2(p["b_g0"]), p["W_nf0"], b2(p["b_nf0"]),
      p["W_ns0"], b2(p["b_ns0"]))

    table1 = pl.pallas_call(
        _scale_body,
        grid=grid,
        in_specs=[_row_spec(CW)] * 10 + [_agg_spec()],
        out_specs=[_row_spec(CW)] * 10,
        out_shape=[jax.ShapeDtypeStruct((NP, CW), f32)] * 10,
    )(*table_u, deg2)

    agg1 = _prop_call(src2d, dst2d, table1, zeros)

    table2_xn = pl.pallas_call(
        _mid_body,
        grid=grid,
        in_specs=[_agg_spec()] * 10 + [_row_spec(CW)] * 10 + [_agg_spec()] +
                 [_full_spec((64, 64)), _full_spec((1, 64)),
                  _full_spec((64, 64)), _full_spec((1, 64)),
                  _full_spec((32, 64)), _full_spec((1, 64))],
        out_specs=[_row_spec(CW)] * 8 + [_row_spec(64)],
        out_shape=[jax.ShapeDtypeStruct((NP, CW), f32)] * 8 +
                  [jax.ShapeDtypeStruct((NP, 64), f32)],
    )(*agg1, *table1, deg2, p["W_g1"], b2(p["b_g1"]),
      p["W_nf1"], b2(p["b_nf1"]), p["W_ns1"], b2(p["b_ns1"]))
    table2 = list(table2_xn[:8])
    xns2 = table2_xn[8]

    agg2 = _prop_call(src2d, dst2d, table2, zeros)

    rp = jnp.stack([
        p["W_r1"][0, 0], p["W_r1"][1, 0], p["b_r1"][0],
        p["W_r1"][0, 1], p["W_r1"][1, 1], p["b_r1"][1],
        p["W_r2"][0, 0], p["W_r2"][1, 0], p["b_r2"][0],
        jnp.float32(0), jnp.float32(0), jnp.float32(0), jnp.float32(0),
        jnp.float32(0), jnp.float32(0), jnp.float32(0)]).reshape(1, 16)

    wfu = p["W_fus"]
    big, small = pl.pallas_call(
        _fin_body,
        grid=grid,
        in_specs=[_row_spec(128), _row_spec(16), _row_spec(16)] +
                 [_agg_spec()] * 8 + [_row_spec(CW)] * 8 +
                 [_agg_spec(), _row_spec(64)] +
                 [_full_spec((128, 64)), _full_spec((1, 64)),
                  _full_spec((16, 64)), _full_spec((1, 64)),
                  _full_spec((16, 64)), _full_spec((1, 64)),
                  _full_spec((64, 64)), _full_spec((1, 64)),
                  _full_spec((64, 64)), _full_spec((1, 64)),
                  _full_spec((64, 64)), _full_spec((64, 64)),
                  _full_spec((64, 64)), _full_spec((64, 64)),
                  _full_spec((1, 64)),
                  _full_spec((64, 32)), _full_spec((1, 32)),
                  _full_spec((32, 1)), _full_spec((1, 1)),
                  _full_spec((64, 32)), _full_spec((1, 32)),
                  _full_spec((32, 1)), _full_spec((1, 1)),
                  _full_spec((1, 16))],
        out_specs=[_row_spec(128), _row_spec(8)],
        out_shape=[jax.ShapeDtypeStruct((NP, 128), f32),
                   jax.ShapeDtypeStruct((NP, 8), f32)],
    )(disc, c1, c2, *agg2, *table2, deg2, xns2,
      p["W_d"], b2(p["b_d"]),
      jnp.pad(p["W_c1"], ((0, 3), (0, 0))), b2(p["b_c1"]),
      jnp.pad(p["W_c2"], ((0, 3), (0, 0))), b2(p["b_c2"]),
      p["W_g2"], b2(p["b_g2"]), p["W_nf2"], b2(p["b_nf2"]),
      wfu[0:64], wfu[64:128], wfu[128:192], wfu[192:256], b2(p["b_fus"]),
      p["W_l1"], b2(p["b_l1"]), p["W_l2"], b2(p["b_l2"]),
      p["W_l3"], b2(p["b_l3"]), p["W_l4"], b2(p["b_l4"]), rp)

    y = small[:NN, 0]
    s_ci = small[:NN, 1:2]
    s_si = small[:NN, 2:3]
    h_ci = big[:NN, :64]
    h_si = big[:NN, 64:]
    return (y, s_ci, s_si, h_ci, h_si)


# speedup vs baseline: 10.6287x; 1.2072x over previous
import jax
import jax.numpy as jnp
from jax import lax
from jax.experimental import pallas as pl
from jax.experimental.pallas import tpu as pltpu
from jax.experimental.pallas import tpu_sc as plsc

NN = 50000
EE = 800000
BT = 1024
NP = 49 * BT
NSC = 2
NTS = 16
RP = NP // NTS
BLK = 1000
NBLK_TOT = EE // BLK
NB0 = 25
NB1 = 25
CW = 16


def _lrelu(x):
    return jnp.where(x > 0, x, x * 0.01)


def _sc_mesh():
    return plsc.VectorSubcoreMesh(core_axis_name="c", subcore_axis_name="s")


def _sc_scratch():
    return [
        pltpu.VMEM((BLK,), jnp.int32),
        pltpu.VMEM((BLK,), jnp.int32),
        pltpu.VMEM((BLK, CW), jnp.float32),
        pltpu.VMEM_SHARED((NP, CW), jnp.float32),
        pltpu.SemaphoreType.DMA,
    ]


def _make_prop_body(C):
    def body(*refs):
        src2d, dst2d = refs[0], refs[1]
        tables = refs[2:2 + C]
        zeros_hbm = refs[2 + C]
        outs = refs[3 + C:3 + 2 * C]
        sidx, didx, rows, acc, gsem = refs[3 + 2 * C:]
        cid = lax.axis_index("c")
        sid = lax.axis_index("s")
        row0 = pl.multiple_of(sid * RP, 8)
        blk0 = jnp.where(cid == 0, sid * NB0, NTS * NB0 + sid * NB1)
        nblk = jnp.where(cid == 0, NB0, NB1)
        for ch in range(C):
            t = tables[ch]
            pltpu.sync_copy(zeros_hbm.at[pl.ds(row0, RP)],
                            acc.at[pl.ds(row0, RP)])
            plsc.subcore_barrier()

            def blk(i, carry, _t=t):
                pltpu.sync_copy(src2d.at[blk0 + i], sidx)
                pltpu.sync_copy(dst2d.at[blk0 + i], didx)
                pltpu.async_copy(_t.at[sidx], rows, gsem).wait()
                pltpu.sync_copy(rows, acc.at[didx], add=True)
                return carry

            lax.fori_loop(0, nblk, blk, 0)
            plsc.subcore_barrier()
            pltpu.sync_copy(acc.at[pl.ds(row0, RP)],
                            outs[ch].at[cid].at[pl.ds(row0, RP)])
            plsc.subcore_barrier()
    return body


def _prop_call(src2d, dst2d, tables, zeros):
    C = len(tables)
    f = pl.kernel(
        _make_prop_body(C),
        out_type=[jax.ShapeDtypeStruct((NSC, NP, CW), jnp.float32)] * C,
        mesh=_sc_mesh(),
        scratch_types=_sc_scratch(),
        compiler_params=pltpu.CompilerParams(use_tc_tiling_on_sc=False),
    )
    out = f(src2d, dst2d, *tables, zeros)
    return out if isinstance(out, (list, tuple)) else [out]


def _dinv_of(deg2):
    deg = deg2[0, :, 0:1] + deg2[1, :, 0:1] + 1.0
    return lax.rsqrt(deg)


def _chunks(x):
    return [x[:, k * CW:(k + 1) * CW] for k in range(x.shape[1] // CW)]


def _pre_body(*refs):
    disc, churn, deg2, wg0, bg0, wf0, bf0, wn0, bn0 = refs[:9]
    outs = refs[9:]
    f32 = jnp.float32
    dinv = _dinv_of(deg2[...])
    d = disc[...]
    xg = _lrelu(jnp.dot(d, wg0[...], preferred_element_type=f32)
                + bg0[...]) * dinv
    xf = _lrelu(jnp.dot(d, wf0[...], preferred_element_type=f32)
                + bf0[...]) * dinv
    xn = _lrelu(jnp.dot(churn[...], wn0[...],
                        preferred_element_type=f32) + bn0[...]) * dinv
    for o, c in zip(outs, _chunks(xg) + _chunks(xf) + _chunks(xn)):
        o[...] = c


def _mid_body(*refs):
    aggs = refs[0:10]
    tabs = refs[10:20]
    deg2, wg1, bg1, wf1, bf1, wn1, bn1 = refs[20:27]
    outs = refs[27:]
    f32 = jnp.float32
    dinv = _dinv_of(deg2[...])
    s = [a[...][0] + a[...][1] + t[...] for a, t in zip(aggs, tabs)]
    tg = dinv * jnp.concatenate(s[0:4], axis=1)
    tf = dinv * jnp.concatenate(s[4:8], axis=1)
    tn = dinv * jnp.concatenate(s[8:10], axis=1)
    xg2 = _lrelu(jnp.dot(tg, wg1[...], preferred_element_type=f32)
                 + bg1[...]) * dinv
    xf2 = _lrelu(jnp.dot(tf, wf1[...], preferred_element_type=f32)
                 + bf1[...]) * dinv
    xn2 = _lrelu(jnp.dot(tn, wn1[...], preferred_element_type=f32)
                 + bn1[...])
    for o, c in zip(outs[:8], _chunks(xg2) + _chunks(xf2)):
        o[...] = c
    outs[8][...] = xn2


def _fin_body(*refs):
    disc, c1, c2 = refs[0:3]
    aggs = refs[3:11]
    tabs = refs[11:19]
    (deg2, xns2, wd, bd, wc1, bc1, wc2, bc2, wg2, bg2, wf2, bf2,
     fu0, fu1, fu2, fu3, bfu, wl1, bl1, wl2, bl2,
     wl3, bl3, wl4, bl4, rp) = refs[19:45]
    obig, osmall = refs[45:]
    f32 = jnp.float32
    dinv = _dinv_of(deg2[...])
    s = [a[...][0] + a[...][1] + t[...] for a, t in zip(aggs, tabs)]
    tg = dinv * jnp.concatenate(s[0:4], axis=1)
    tf = dinv * jnp.concatenate(s[4:8], axis=1)
    xg3 = _lrelu(jnp.dot(tg, wg2[...], preferred_element_type=f32) + bg2[...])
    xf3 = _lrelu(jnp.dot(tf, wf2[...], preferred_element_type=f32) + bf2[...])
    d = disc[...]
    xd = _lrelu(jnp.dot(d, wd[...], preferred_element_type=f32) + bd[...])
    xc1 = _lrelu(jnp.dot(c1[...], wc1[...], preferred_element_type=f32)
                 + bc1[...])
    xc2 = _lrelu(jnp.dot(c2[...], wc2[...], preferred_element_type=f32)
                 + bc2[...])
    h_ci = _lrelu(jnp.dot(xd, fu0[...], preferred_element_type=f32)
                  + jnp.dot(xc1, fu1[...], preferred_element_type=f32)
                  + jnp.dot(xc2, fu2[...], preferred_element_type=f32)
                  + jnp.dot(xg3, fu3[...], preferred_element_type=f32)
                  + bfu[...])
    h_si = xf3 * xns2[...]
    s_ci = jax.nn.sigmoid(
        jnp.dot(_lrelu(jnp.dot(h_ci, wl1[...], preferred_element_type=f32)
                       + bl1[...]), wl2[...], preferred_element_type=f32)
        + bl2[...])
    s_si = jax.nn.sigmoid(
        jnp.dot(_lrelu(jnp.dot(h_si, wl3[...], preferred_element_type=f32)
                       + bl3[...]), wl4[...], preferred_element_type=f32)
        + bl4[...])
    r = rp[...]
    r1a = _lrelu(s_ci * r[0:1, 0:1] + s_si * r[0:1, 1:2] + r[0:1, 2:3])
    r1b = _lrelu(s_ci * r[0:1, 3:4] + s_si * r[0:1, 4:5] + r[0:1, 5:6])
    y = jax.nn.sigmoid(r1a * r[0:1, 6:7] + r1b * r[0:1, 7:8] + r[0:1, 8:9])
    obig[...] = jnp.concatenate([h_ci, h_si], axis=1)
    z = jnp.zeros_like(y)
    osmall[...] = jnp.concatenate([y, s_ci, s_si, z, z, z, z, z], axis=1)


def _row_spec(cols):
    return pl.BlockSpec((BT, cols), lambda i: (i, 0))


def _full_spec(shape):
    nd = len(shape)
    return pl.BlockSpec(shape, lambda i: (0,) * nd)


def _agg_spec():
    return pl.BlockSpec((NSC, BT, CW), lambda i: (0, i, 0))


def _pad_rows(x):
    return jnp.pad(x, ((0, NP - x.shape[0]), (0, 0)))


def kernel(discrete_x, continous_x, churn_date, edge_index, edge_attr, params):
    p = params
    f32 = jnp.float32
    src2d = edge_index[0].reshape(NBLK_TOT, BLK)
    dst2d = edge_index[1].reshape(NBLK_TOT, BLK)
    disc = _pad_rows(discrete_x)
    churn = _pad_rows(churn_date)
    c1 = _pad_rows(jnp.pad(continous_x[:, :13], ((0, 0), (0, 3))))
    c2 = _pad_rows(jnp.pad(continous_x[:, 13:26], ((0, 0), (0, 3))))
    zeros = jnp.zeros((NP, CW), f32)
    ones = jnp.ones((NP, CW), f32)

    def b2(b):
        return b.reshape(1, -1)

    deg2 = _prop_call(src2d, dst2d, [ones], zeros)[0]

    grid = (NP // BT,)
    table1 = pl.pallas_call(
        _pre_body,
        grid=grid,
        in_specs=[_row_spec(128), _row_spec(8), _agg_spec(),
                  _full_spec((128, 64)), _full_spec((1, 64)),
                  _full_spec((128, 64)), _full_spec((1, 64)),
                  _full_spec((8, 32)), _full_spec((1, 32))],
        out_specs=[_row_spec(CW)] * 10,
        out_shape=[jax.ShapeDtypeStruct((NP, CW), f32)] * 10,
    )(disc, churn, deg2, p["W_g0"], b2(p["b_g0"]),
      p["W_nf0"], b2(p["b_nf0"]), p["W_ns0"], b2(p["b_ns0"]))

    agg1 = _prop_call(src2d, dst2d, table1, zeros)

    table2_xn = pl.pallas_call(
        _mid_body,
        grid=grid,
        in_specs=[_agg_spec()] * 10 + [_row_spec(CW)] * 10 + [_agg_spec()] +
                 [_full_spec((64, 64)), _full_spec((1, 64)),
                  _full_spec((64, 64)), _full_spec((1, 64)),
                  _full_spec((32, 64)), _full_spec((1, 64))],
        out_specs=[_row_spec(CW)] * 8 + [_row_spec(64)],
        out_shape=[jax.ShapeDtypeStruct((NP, CW), f32)] * 8 +
                  [jax.ShapeDtypeStruct((NP, 64), f32)],
    )(*agg1, *table1, deg2, p["W_g1"], b2(p["b_g1"]),
      p["W_nf1"], b2(p["b_nf1"]), p["W_ns1"], b2(p["b_ns1"]))
    table2 = list(table2_xn[:8])
    xns2 = table2_xn[8]

    agg2 = _prop_call(src2d, dst2d, table2, zeros)

    rp = jnp.stack([
        p["W_r1"][0, 0], p["W_r1"][1, 0], p["b_r1"][0],
        p["W_r1"][0, 1], p["W_r1"][1, 1], p["b_r1"][1],
        p["W_r2"][0, 0], p["W_r2"][1, 0], p["b_r2"][0],
        jnp.float32(0), jnp.float32(0), jnp.float32(0), jnp.float32(0),
        jnp.float32(0), jnp.float32(0), jnp.float32(0)]).reshape(1, 16)

    wfu = p["W_fus"]
    big, small = pl.pallas_call(
        _fin_body,
        grid=grid,
        in_specs=[_row_spec(128), _row_spec(16), _row_spec(16)] +
                 [_agg_spec()] * 8 + [_row_spec(CW)] * 8 +
                 [_agg_spec(), _row_spec(64)] +
                 [_full_spec((128, 64)), _full_spec((1, 64)),
                  _full_spec((16, 64)), _full_spec((1, 64)),
                  _full_spec((16, 64)), _full_spec((1, 64)),
                  _full_spec((64, 64)), _full_spec((1, 64)),
                  _full_spec((64, 64)), _full_spec((1, 64)),
                  _full_spec((64, 64)), _full_spec((64, 64)),
                  _full_spec((64, 64)), _full_spec((64, 64)),
                  _full_spec((1, 64)),
                  _full_spec((64, 32)), _full_spec((1, 32)),
                  _full_spec((32, 1)), _full_spec((1, 1)),
                  _full_spec((64, 32)), _full_spec((1, 32)),
                  _full_spec((32, 1)), _full_spec((1, 1)),
                  _full_spec((1, 16))],
        out_specs=[_row_spec(128), _row_spec(8)],
        out_shape=[jax.ShapeDtypeStruct((NP, 128), f32),
                   jax.ShapeDtypeStruct((NP, 8), f32)],
    )(disc, c1, c2, *agg2, *table2, deg2, xns2,
      p["W_d"], b2(p["b_d"]),
      jnp.pad(p["W_c1"], ((0, 3), (0, 0))), b2(p["b_c1"]),
      jnp.pad(p["W_c2"], ((0, 3), (0, 0))), b2(p["b_c2"]),
      p["W_g2"], b2(p["b_g2"]), p["W_nf2"], b2(p["b_nf2"]),
      wfu[0:64], wfu[64:128], wfu[128:192], wfu[192:256], b2(p["b_fus"]),
      p["W_l1"], b2(p["b_l1"]), p["W_l2"], b2(p["b_l2"]),
      p["W_l3"], b2(p["b_l3"]), p["W_l4"], b2(p["b_l4"]), rp)

    y = small[:NN, 0]
    s_ci = small[:NN, 1:2]
    s_si = small[:NN, 2:3]
    h_ci = big[:NN, :64]
    h_si = big[:NN, 64:]
    return (y, s_ci, s_si, h_ci, h_si)


# speedup vs baseline: 13.4408x; 1.2646x over previous
import jax
import jax.numpy as jnp
from jax import lax
from jax.experimental import pallas as pl
from jax.experimental.pallas import tpu as pltpu
from jax.experimental.pallas import tpu_sc as plsc

NN = 50000
EE = 800000
BT = 1024
NP = 49 * BT
NSC = 2
NTS = 16
RP = NP // NTS
BLK = 1000
NBLK_TOT = EE // BLK
NB0 = 25
NB1 = 25
CW = 16


def _lrelu(x):
    return jnp.where(x > 0, x, x * 0.01)


def _sc_mesh():
    return plsc.VectorSubcoreMesh(core_axis_name="c", subcore_axis_name="s")


def _sc_scratch():
    return [
        pltpu.VMEM((BLK,), jnp.int32),
        pltpu.VMEM((BLK,), jnp.int32),
        pltpu.VMEM((BLK,), jnp.int32),
        pltpu.VMEM((BLK, CW), jnp.float32),
        pltpu.VMEM((BLK, CW), jnp.float32),
        pltpu.VMEM_SHARED((NP, CW), jnp.float32),
        pltpu.SemaphoreType.DMA,
        pltpu.SemaphoreType.DMA,
    ]


NBLK = NB0


def _make_prop_body(C):
    def body(*refs):
        src2d, dst2d = refs[0], refs[1]
        tables = refs[2:2 + C]
        zeros_hbm = refs[2 + C]
        outs = refs[3 + C:3 + 2 * C]
        sidx0, sidx1, didx, rows0, rows1, acc, gsem0, gsem1 = \
            refs[3 + 2 * C:]
        sidx = (sidx0, sidx1)
        rows = (rows0, rows1)
        gsem = (gsem0, gsem1)
        cid = lax.axis_index("c")
        sid = lax.axis_index("s")
        row0 = pl.multiple_of(sid * RP, 8)
        blk0 = (cid * NTS + sid) * NBLK
        for ch in range(C):
            t = tables[ch]
            pltpu.sync_copy(zeros_hbm.at[pl.ds(row0, RP)],
                            acc.at[pl.ds(row0, RP)])
            plsc.subcore_barrier()
            pltpu.sync_copy(src2d.at[blk0], sidx[0])
            pltpu.async_copy(t.at[sidx[0]], rows[0], gsem[0])

            def step(g, b, _t, last):
                if not last:
                    pltpu.sync_copy(src2d.at[blk0 + g + 1], sidx[1 - b])
                    pltpu.async_copy(_t.at[sidx[1 - b]], rows[1 - b],
                                     gsem[1 - b])
                pltpu.sync_copy(dst2d.at[blk0 + g], didx)
                pltpu.make_async_copy(_t.at[sidx[b]], rows[b],
                                      gsem[b]).wait()
                pltpu.sync_copy(rows[b], acc.at[didx], add=True)

            def pair(j, carry, _t=t):
                step(2 * j, 0, _t, False)
                step(2 * j + 1, 1, _t, False)
                return carry

            lax.fori_loop(0, (NBLK - 1) // 2, pair, 0)
            step(NBLK - 1, (NBLK - 1) % 2, t, True)
            plsc.subcore_barrier()
            pltpu.sync_copy(acc.at[pl.ds(row0, RP)],
                            outs[ch].at[cid].at[pl.ds(row0, RP)])
            plsc.subcore_barrier()
    return body


def _prop_call(src2d, dst2d, tables, zeros):
    C = len(tables)
    f = pl.kernel(
        _make_prop_body(C),
        out_type=[jax.ShapeDtypeStruct((NSC, NP, CW), jnp.float32)] * C,
        mesh=_sc_mesh(),
        scratch_types=_sc_scratch(),
        compiler_params=pltpu.CompilerParams(use_tc_tiling_on_sc=False),
    )
    out = f(src2d, dst2d, *tables, zeros)
    return out if isinstance(out, (list, tuple)) else [out]


def _dinv_of(deg2):
    deg = deg2[0, :, 0:1] + deg2[1, :, 0:1] + 1.0
    return lax.rsqrt(deg)


def _chunks(x):
    return [x[:, k * CW:(k + 1) * CW] for k in range(x.shape[1] // CW)]


def _pre_body(*refs):
    disc, churn, deg2, wg0, bg0, wf0, bf0, wn0, bn0 = refs[:9]
    outs = refs[9:]
    f32 = jnp.float32
    dinv = _dinv_of(deg2[...])
    d = disc[...]
    xg = _lrelu(jnp.dot(d, wg0[...], preferred_element_type=f32)
                + bg0[...]) * dinv
    xf = _lrelu(jnp.dot(d, wf0[...], preferred_element_type=f32)
                + bf0[...]) * dinv
    xn = _lrelu(jnp.dot(churn[...], wn0[...],
                        preferred_element_type=f32) + bn0[...]) * dinv
    for o, c in zip(outs, _chunks(xg) + _chunks(xf) + _chunks(xn)):
        o[...] = c


def _mid_body(*refs):
    aggs = refs[0:10]
    tabs = refs[10:20]
    deg2, wg1, bg1, wf1, bf1, wn1, bn1 = refs[20:27]
    outs = refs[27:]
    f32 = jnp.float32
    dinv = _dinv_of(deg2[...])
    s = [a[...][0] + a[...][1] + t[...] for a, t in zip(aggs, tabs)]
    tg = dinv * jnp.concatenate(s[0:4], axis=1)
    tf = dinv * jnp.concatenate(s[4:8], axis=1)
    tn = dinv * jnp.concatenate(s[8:10], axis=1)
    xg2 = _lrelu(jnp.dot(tg, wg1[...], preferred_element_type=f32)
                 + bg1[...]) * dinv
    xf2 = _lrelu(jnp.dot(tf, wf1[...], preferred_element_type=f32)
                 + bf1[...]) * dinv
    xn2 = _lrelu(jnp.dot(tn, wn1[...], preferred_element_type=f32)
                 + bn1[...])
    for o, c in zip(outs[:8], _chunks(xg2) + _chunks(xf2)):
        o[...] = c
    outs[8][...] = xn2


def _fin_body(*refs):
    disc, c1, c2 = refs[0:3]
    aggs = refs[3:11]
    tabs = refs[11:19]
    (deg2, xns2, wd, bd, wc1, bc1, wc2, bc2, wg2, bg2, wf2, bf2,
     fu0, fu1, fu2, fu3, bfu, wl1, bl1, wl2, bl2,
     wl3, bl3, wl4, bl4, rp) = refs[19:45]
    obig, osmall = refs[45:]
    f32 = jnp.float32
    dinv = _dinv_of(deg2[...])
    s = [a[...][0] + a[...][1] + t[...] for a, t in zip(aggs, tabs)]
    tg = dinv * jnp.concatenate(s[0:4], axis=1)
    tf = dinv * jnp.concatenate(s[4:8], axis=1)
    xg3 = _lrelu(jnp.dot(tg, wg2[...], preferred_element_type=f32) + bg2[...])
    xf3 = _lrelu(jnp.dot(tf, wf2[...], preferred_element_type=f32) + bf2[...])
    d = disc[...]
    xd = _lrelu(jnp.dot(d, wd[...], preferred_element_type=f32) + bd[...])
    xc1 = _lrelu(jnp.dot(c1[...], wc1[...], preferred_element_type=f32)
                 + bc1[...])
    xc2 = _lrelu(jnp.dot(c2[...], wc2[...], preferred_element_type=f32)
                 + bc2[...])
    h_ci = _lrelu(jnp.dot(xd, fu0[...], preferred_element_type=f32)
                  + jnp.dot(xc1, fu1[...], preferred_element_type=f32)
                  + jnp.dot(xc2, fu2[...], preferred_element_type=f32)
                  + jnp.dot(xg3, fu3[...], preferred_element_type=f32)
                  + bfu[...])
    h_si = xf3 * xns2[...]
    s_ci = jax.nn.sigmoid(
        jnp.dot(_lrelu(jnp.dot(h_ci, wl1[...], preferred_element_type=f32)
                       + bl1[...]), wl2[...], preferred_element_type=f32)
        + bl2[...])
    s_si = jax.nn.sigmoid(
        jnp.dot(_lrelu(jnp.dot(h_si, wl3[...], preferred_element_type=f32)
                       + bl3[...]), wl4[...], preferred_element_type=f32)
        + bl4[...])
    r = rp[...]
    r1a = _lrelu(s_ci * r[0:1, 0:1] + s_si * r[0:1, 1:2] + r[0:1, 2:3])
    r1b = _lrelu(s_ci * r[0:1, 3:4] + s_si * r[0:1, 4:5] + r[0:1, 5:6])
    y = jax.nn.sigmoid(r1a * r[0:1, 6:7] + r1b * r[0:1, 7:8] + r[0:1, 8:9])
    obig[...] = jnp.concatenate([h_ci, h_si], axis=1)
    z = jnp.zeros_like(y)
    osmall[...] = jnp.concatenate([y, s_ci, s_si, z, z, z, z, z], axis=1)


def _row_spec(cols):
    return pl.BlockSpec((BT, cols), lambda i: (i, 0))


def _full_spec(shape):
    nd = len(shape)
    return pl.BlockSpec(shape, lambda i: (0,) * nd)


def _agg_spec():
    return pl.BlockSpec((NSC, BT, CW), lambda i: (0, i, 0))


def _pad_rows(x):
    return jnp.pad(x, ((0, NP - x.shape[0]), (0, 0)))


def kernel(discrete_x, continous_x, churn_date, edge_index, edge_attr, params):
    p = params
    f32 = jnp.float32
    src2d = edge_index[0].reshape(NBLK_TOT, BLK)
    dst2d = edge_index[1].reshape(NBLK_TOT, BLK)
    disc = _pad_rows(discrete_x)
    churn = _pad_rows(churn_date)
    c1 = _pad_rows(jnp.pad(continous_x[:, :13], ((0, 0), (0, 3))))
    c2 = _pad_rows(jnp.pad(continous_x[:, 13:26], ((0, 0), (0, 3))))
    zeros = jnp.zeros((NP, CW), f32)
    ones = jnp.ones((NP, CW), f32)

    def b2(b):
        return b.reshape(1, -1)

    deg2 = _prop_call(src2d, dst2d, [ones], zeros)[0]

    grid = (NP // BT,)
    table1 = pl.pallas_call(
        _pre_body,
        grid=grid,
        in_specs=[_row_spec(128), _row_spec(8), _agg_spec(),
                  _full_spec((128, 64)), _full_spec((1, 64)),
                  _full_spec((128, 64)), _full_spec((1, 64)),
                  _full_spec((8, 32)), _full_spec((1, 32))],
        out_specs=[_row_spec(CW)] * 10,
        out_shape=[jax.ShapeDtypeStruct((NP, CW), f32)] * 10,
    )(disc, churn, deg2, p["W_g0"], b2(p["b_g0"]),
      p["W_nf0"], b2(p["b_nf0"]), p["W_ns0"], b2(p["b_ns0"]))

    agg1 = _prop_call(src2d, dst2d, table1, zeros)

    table2_xn = pl.pallas_call(
        _mid_body,
        grid=grid,
        in_specs=[_agg_spec()] * 10 + [_row_spec(CW)] * 10 + [_agg_spec()] +
                 [_full_spec((64, 64)), _full_spec((1, 64)),
                  _full_spec((64, 64)), _full_spec((1, 64)),
                  _full_spec((32, 64)), _full_spec((1, 64))],
        out_specs=[_row_spec(CW)] * 8 + [_row_spec(64)],
        out_shape=[jax.ShapeDtypeStruct((NP, CW), f32)] * 8 +
                  [jax.ShapeDtypeStruct((NP, 64), f32)],
    )(*agg1, *table1, deg2, p["W_g1"], b2(p["b_g1"]),
      p["W_nf1"], b2(p["b_nf1"]), p["W_ns1"], b2(p["b_ns1"]))
    table2 = list(table2_xn[:8])
    xns2 = table2_xn[8]

    agg2 = _prop_call(src2d, dst2d, table2, zeros)

    rp = jnp.stack([
        p["W_r1"][0, 0], p["W_r1"][1, 0], p["b_r1"][0],
        p["W_r1"][0, 1], p["W_r1"][1, 1], p["b_r1"][1],
        p["W_r2"][0, 0], p["W_r2"][1, 0], p["b_r2"][0],
        jnp.float32(0), jnp.float32(0), jnp.float32(0), jnp.float32(0),
        jnp.float32(0), jnp.float32(0), jnp.float32(0)]).reshape(1, 16)

    wfu = p["W_fus"]
    big, small = pl.pallas_call(
        _fin_body,
        grid=grid,
        in_specs=[_row_spec(128), _row_spec(16), _row_spec(16)] +
                 [_agg_spec()] * 8 + [_row_spec(CW)] * 8 +
                 [_agg_spec(), _row_spec(64)] +
                 [_full_spec((128, 64)), _full_spec((1, 64)),
                  _full_spec((16, 64)), _full_spec((1, 64)),
                  _full_spec((16, 64)), _full_spec((1, 64)),
                  _full_spec((64, 64)), _full_spec((1, 64)),
                  _full_spec((64, 64)), _full_spec((1, 64)),
                  _full_spec((64, 64)), _full_spec((64, 64)),
                  _full_spec((64, 64)), _full_spec((64, 64)),
                  _full_spec((1, 64)),
                  _full_spec((64, 32)), _full_spec((1, 32)),
                  _full_spec((32, 1)), _full_spec((1, 1)),
                  _full_spec((64, 32)), _full_spec((1, 32)),
                  _full_spec((32, 1)), _full_spec((1, 1)),
                  _full_spec((1, 16))],
        out_specs=[_row_spec(128), _row_spec(8)],
        out_shape=[jax.ShapeDtypeStruct((NP, 128), f32),
                   jax.ShapeDtypeStruct((NP, 8), f32)],
    )(disc, c1, c2, *agg2, *table2, deg2, xns2,
      p["W_d"], b2(p["b_d"]),
      jnp.pad(p["W_c1"], ((0, 3), (0, 0))), b2(p["b_c1"]),
      jnp.pad(p["W_c2"], ((0, 3), (0, 0))), b2(p["b_c2"]),
      p["W_g2"], b2(p["b_g2"]), p["W_nf2"], b2(p["b_nf2"]),
      wfu[0:64], wfu[64:128], wfu[128:192], wfu[192:256], b2(p["b_fus"]),
      p["W_l1"], b2(p["b_l1"]), p["W_l2"], b2(p["b_l2"]),
      p["W_l3"], b2(p["b_l3"]), p["W_l4"], b2(p["b_l4"]), rp)

    y = small[:NN, 0]
    s_ci = small[:NN, 1:2]
    s_si = small[:NN, 2:3]
    h_ci = big[:NN, :64]
    h_si = big[:NN, 64:]
    return (y, s_ci, s_si, h_ci, h_si)


# speedup vs baseline: 15.5184x; 1.1546x over previous
import jax
import jax.numpy as jnp
from jax import lax
from jax.experimental import pallas as pl
from jax.experimental.pallas import tpu as pltpu
from jax.experimental.pallas import tpu_sc as plsc

NN = 50000
EE = 800000
BT = 1024
NP = 49 * BT
NSC = 2
NTS = 16
RP = NP // NTS
BLK = 1000
NBLK_TOT = EE // BLK
NB0 = 25
NB1 = 25
CW = 16


def _lrelu(x):
    return jnp.where(x > 0, x, x * 0.01)


def _sc_mesh():
    return plsc.VectorSubcoreMesh(core_axis_name="c", subcore_axis_name="s")


def _sc_scratch():
    return [
        pltpu.VMEM((BLK,), jnp.int32),
        pltpu.VMEM((BLK,), jnp.int32),
        pltpu.VMEM((BLK,), jnp.int32),
        pltpu.VMEM((BLK, CW), jnp.float32),
        pltpu.VMEM((BLK, CW), jnp.float32),
        pltpu.VMEM_SHARED((NP, CW), jnp.float32),
        pltpu.SemaphoreType.DMA,
        pltpu.SemaphoreType.DMA,
    ]


NBLK = NB0


def _make_prop_body(C):
    def body(*refs):
        src2d, dst2d = refs[0], refs[1]
        tables = refs[2:2 + C]
        zeros_hbm = refs[2 + C]
        out = refs[3 + C]
        sidx0, sidx1, didx, rows0, rows1, acc, gsem0, gsem1 = \
            refs[4 + C:]
        sidx = (sidx0, sidx1)
        rows = (rows0, rows1)
        gsem = (gsem0, gsem1)
        cid = lax.axis_index("c")
        sid = lax.axis_index("s")
        row0 = pl.multiple_of(sid * RP, 8)
        blk0 = (cid * NTS + sid) * NBLK
        for ch in range(C):
            t = tables[ch]
            pltpu.sync_copy(zeros_hbm.at[pl.ds(row0, RP)],
                            acc.at[pl.ds(row0, RP)])
            plsc.subcore_barrier()
            pltpu.sync_copy(src2d.at[blk0], sidx[0])
            pltpu.async_copy(t.at[sidx[0]], rows[0], gsem[0])

            def step(g, b, _t, last):
                if not last:
                    pltpu.sync_copy(src2d.at[blk0 + g + 1], sidx[1 - b])
                    pltpu.async_copy(_t.at[sidx[1 - b]], rows[1 - b],
                                     gsem[1 - b])
                pltpu.sync_copy(dst2d.at[blk0 + g], didx)
                pltpu.make_async_copy(_t.at[sidx[b]], rows[b],
                                      gsem[b]).wait()
                pltpu.sync_copy(rows[b], acc.at[didx], add=True)

            def pair(j, carry, _t=t):
                step(2 * j, 0, _t, False)
                step(2 * j + 1, 1, _t, False)
                return carry

            lax.fori_loop(0, (NBLK - 1) // 2, pair, 0)
            step(NBLK - 1, (NBLK - 1) % 2, t, True)
            plsc.subcore_barrier()
            if C == 1:
                dst = out.at[cid].at[pl.ds(row0, RP)]
            else:
                dst = out.at[cid].at[pl.ds(row0, RP), pl.ds(ch * CW, CW)]
            pltpu.sync_copy(acc.at[pl.ds(row0, RP)], dst)
            plsc.subcore_barrier()
    return body


def _prop_call(src2d, dst2d, tables, zeros):
    C = len(tables)
    f = pl.kernel(
        _make_prop_body(C),
        out_type=jax.ShapeDtypeStruct((NSC, NP, CW * C), jnp.float32),
        mesh=_sc_mesh(),
        scratch_types=_sc_scratch(),
        compiler_params=pltpu.CompilerParams(use_tc_tiling_on_sc=False),
    )
    return f(src2d, dst2d, *tables, zeros)


def _dinv_of(deg2):
    deg = deg2[0, :, 0:1] + deg2[1, :, 0:1] + 1.0
    return lax.rsqrt(deg)


def _chunks(x):
    return [x[:, k * CW:(k + 1) * CW] for k in range(x.shape[1] // CW)]


def _pre_body(*refs):
    disc, churn, deg2, wg0, bg0, wf0, bf0, wn0, bn0 = refs[:9]
    outs = refs[9:]
    f32 = jnp.float32
    dinv = _dinv_of(deg2[...])
    d = disc[...]
    xg = _lrelu(jnp.dot(d, wg0[...], preferred_element_type=f32)
                + bg0[...]) * dinv
    xf = _lrelu(jnp.dot(d, wf0[...], preferred_element_type=f32)
                + bf0[...]) * dinv
    xn = _lrelu(jnp.dot(churn[...], wn0[...],
                        preferred_element_type=f32) + bn0[...]) * dinv
    for o, c in zip(outs, _chunks(xg) + _chunks(xf) + _chunks(xn)):
        o[...] = c


def _mid_body(*refs):
    aw = refs[0]
    tabs = refs[1:11]
    deg2, wg1, bg1, wf1, bf1, wn1, bn1 = refs[11:18]
    outs = refs[18:]
    f32 = jnp.float32
    dinv = _dinv_of(deg2[...])
    a = aw[...][0] + aw[...][1]
    sw = a + jnp.concatenate([t[...] for t in tabs], axis=1)
    tg = dinv * sw[:, 0:64]
    tf = dinv * sw[:, 64:128]
    tn = dinv * sw[:, 128:160]
    xg2 = _lrelu(jnp.dot(tg, wg1[...], preferred_element_type=f32)
                 + bg1[...]) * dinv
    xf2 = _lrelu(jnp.dot(tf, wf1[...], preferred_element_type=f32)
                 + bf1[...]) * dinv
    xn2 = _lrelu(jnp.dot(tn, wn1[...], preferred_element_type=f32)
                 + bn1[...])
    for o, c in zip(outs[:8], _chunks(xg2) + _chunks(xf2)):
        o[...] = c
    outs[8][...] = xn2


def _fin_body(*refs):
    disc, c1, c2 = refs[0:3]
    aw = refs[3]
    tabs = refs[4:12]
    (deg2, xns2, wd, bd, wc1, bc1, wc2, bc2, wg2, bg2, wf2, bf2,
     fu0, fu1, fu2, fu3, bfu, wl1, bl1, wl2, bl2,
     wl3, bl3, wl4, bl4, rp) = refs[12:38]
    obig, osmall = refs[38:]
    f32 = jnp.float32
    dinv = _dinv_of(deg2[...])
    a = aw[...][0] + aw[...][1]
    sw = a + jnp.concatenate([t[...] for t in tabs], axis=1)
    tg = dinv * sw[:, 0:64]
    tf = dinv * sw[:, 64:128]
    xg3 = _lrelu(jnp.dot(tg, wg2[...], preferred_element_type=f32) + bg2[...])
    xf3 = _lrelu(jnp.dot(tf, wf2[...], preferred_element_type=f32) + bf2[...])
    d = disc[...]
    xd = _lrelu(jnp.dot(d, wd[...], preferred_element_type=f32) + bd[...])
    xc1 = _lrelu(jnp.dot(c1[...], wc1[...], preferred_element_type=f32)
                 + bc1[...])
    xc2 = _lrelu(jnp.dot(c2[...], wc2[...], preferred_element_type=f32)
                 + bc2[...])
    h_ci = _lrelu(jnp.dot(xd, fu0[...], preferred_element_type=f32)
                  + jnp.dot(xc1, fu1[...], preferred_element_type=f32)
                  + jnp.dot(xc2, fu2[...], preferred_element_type=f32)
                  + jnp.dot(xg3, fu3[...], preferred_element_type=f32)
                  + bfu[...])
    h_si = xf3 * xns2[...]
    s_ci = jax.nn.sigmoid(
        jnp.dot(_lrelu(jnp.dot(h_ci, wl1[...], preferred_element_type=f32)
                       + bl1[...]), wl2[...], preferred_element_type=f32)
        + bl2[...])
    s_si = jax.nn.sigmoid(
        jnp.dot(_lrelu(jnp.dot(h_si, wl3[...], preferred_element_type=f32)
                       + bl3[...]), wl4[...], preferred_element_type=f32)
        + bl4[...])
    r = rp[...]
    r1a = _lrelu(s_ci * r[0:1, 0:1] + s_si * r[0:1, 1:2] + r[0:1, 2:3])
    r1b = _lrelu(s_ci * r[0:1, 3:4] + s_si * r[0:1, 4:5] + r[0:1, 5:6])
    y = jax.nn.sigmoid(r1a * r[0:1, 6:7] + r1b * r[0:1, 7:8] + r[0:1, 8:9])
    obig[...] = jnp.concatenate([h_ci, h_si], axis=1)
    z = jnp.zeros_like(y)
    osmall[...] = jnp.concatenate([y, s_ci, s_si, z, z, z, z, z], axis=1)


def _row_spec(cols):
    return pl.BlockSpec((BT, cols), lambda i: (i, 0))


def _full_spec(shape):
    nd = len(shape)
    return pl.BlockSpec(shape, lambda i: (0,) * nd)


def _agg_spec(cols):
    return pl.BlockSpec((NSC, BT, cols), lambda i: (0, i, 0))


def _pad_rows(x):
    return jnp.pad(x, ((0, NP - x.shape[0]), (0, 0)))


def kernel(discrete_x, continous_x, churn_date, edge_index, edge_attr, params):
    p = params
    f32 = jnp.float32
    src2d = edge_index[0].reshape(NBLK_TOT, BLK)
    dst2d = edge_index[1].reshape(NBLK_TOT, BLK)
    disc = _pad_rows(discrete_x)
    churn = _pad_rows(churn_date)
    c1 = _pad_rows(jnp.pad(continous_x[:, :13], ((0, 0), (0, 3))))
    c2 = _pad_rows(jnp.pad(continous_x[:, 13:26], ((0, 0), (0, 3))))
    zeros = jnp.zeros((NP, CW), f32)
    ones = jnp.ones((NP, CW), f32)

    def b2(b):
        return b.reshape(1, -1)

    deg2 = _prop_call(src2d, dst2d, [ones], zeros)

    grid = (NP // BT,)
    table1 = pl.pallas_call(
        _pre_body,
        grid=grid,
        in_specs=[_row_spec(128), _row_spec(8), _agg_spec(CW),
                  _full_spec((128, 64)), _full_spec((1, 64)),
                  _full_spec((128, 64)), _full_spec((1, 64)),
                  _full_spec((8, 32)), _full_spec((1, 32))],
        out_specs=[_row_spec(CW)] * 10,
        out_shape=[jax.ShapeDtypeStruct((NP, CW), f32)] * 10,
    )(disc, churn, deg2, p["W_g0"], b2(p["b_g0"]),
      p["W_nf0"], b2(p["b_nf0"]), p["W_ns0"], b2(p["b_ns0"]))

    agg1 = _prop_call(src2d, dst2d, table1, zeros)

    table2_xn = pl.pallas_call(
        _mid_body,
        grid=grid,
        in_specs=[_agg_spec(160)] + [_row_spec(CW)] * 10 + [_agg_spec(CW)] +
                 [_full_spec((64, 64)), _full_spec((1, 64)),
                  _full_spec((64, 64)), _full_spec((1, 64)),
                  _full_spec((32, 64)), _full_spec((1, 64))],
        out_specs=[_row_spec(CW)] * 8 + [_row_spec(64)],
        out_shape=[jax.ShapeDtypeStruct((NP, CW), f32)] * 8 +
                  [jax.ShapeDtypeStruct((NP, 64), f32)],
    )(agg1, *table1, deg2, p["W_g1"], b2(p["b_g1"]),
      p["W_nf1"], b2(p["b_nf1"]), p["W_ns1"], b2(p["b_ns1"]))
    table2 = list(table2_xn[:8])
    xns2 = table2_xn[8]

    agg2 = _prop_call(src2d, dst2d, table2, zeros)

    rp = jnp.stack([
        p["W_r1"][0, 0], p["W_r1"][1, 0], p["b_r1"][0],
        p["W_r1"][0, 1], p["W_r1"][1, 1], p["b_r1"][1],
        p["W_r2"][0, 0], p["W_r2"][1, 0], p["b_r2"][0],
        jnp.float32(0), jnp.float32(0), jnp.float32(0), jnp.float32(0),
        jnp.float32(0), jnp.float32(0), jnp.float32(0)]).reshape(1, 16)

    wfu = p["W_fus"]
    big, small = pl.pallas_call(
        _fin_body,
        grid=grid,
        in_specs=[_row_spec(128), _row_spec(16), _row_spec(16)] +
                 [_agg_spec(128)] + [_row_spec(CW)] * 8 +
                 [_agg_spec(CW), _row_spec(64)] +
                 [_full_spec((128, 64)), _full_spec((1, 64)),
                  _full_spec((16, 64)), _full_spec((1, 64)),
                  _full_spec((16, 64)), _full_spec((1, 64)),
                  _full_spec((64, 64)), _full_spec((1, 64)),
                  _full_spec((64, 64)), _full_spec((1, 64)),
                  _full_spec((64, 64)), _full_spec((64, 64)),
                  _full_spec((64, 64)), _full_spec((64, 64)),
                  _full_spec((1, 64)),
                  _full_spec((64, 32)), _full_spec((1, 32)),
                  _full_spec((32, 1)), _full_spec((1, 1)),
                  _full_spec((64, 32)), _full_spec((1, 32)),
                  _full_spec((32, 1)), _full_spec((1, 1)),
                  _full_spec((1, 16))],
        out_specs=[_row_spec(128), _row_spec(8)],
        out_shape=[jax.ShapeDtypeStruct((NP, 128), f32),
                   jax.ShapeDtypeStruct((NP, 8), f32)],
    )(disc, c1, c2, agg2, *table2, deg2, xns2,
      p["W_d"], b2(p["b_d"]),
      jnp.pad(p["W_c1"], ((0, 3), (0, 0))), b2(p["b_c1"]),
      jnp.pad(p["W_c2"], ((0, 3), (0, 0))), b2(p["b_c2"]),
      p["W_g2"], b2(p["b_g2"]), p["W_nf2"], b2(p["b_nf2"]),
      wfu[0:64], wfu[64:128], wfu[128:192], wfu[192:256], b2(p["b_fus"]),
      p["W_l1"], b2(p["b_l1"]), p["W_l2"], b2(p["b_l2"]),
      p["W_l3"], b2(p["b_l3"]), p["W_l4"], b2(p["b_l4"]), rp)

    y = small[:NN, 0]
    s_ci = small[:NN, 1:2]
    s_si = small[:NN, 2:3]
    h_ci = big[:NN, :64]
    h_si = big[:NN, 64:]
    return (y, s_ci, s_si, h_ci, h_si)
